# baseline (speedup 1.0000x reference)
import math

import jax
import jax.numpy as jnp
from jax import lax
from jax.experimental import pallas as pl
from jax.experimental.pallas import tpu as pltpu
from jax.experimental.pallas import tpu_sc as plsc

D_MODEL = 64
SCALE = math.sqrt(D_MODEL)

NC = 2
NS = 16
NW = NC * NS

C = 128
NBUF = 4
ROW_UNROLL = 8


def _scale_chunk(src, dst):
    def body(i, carry):
        for rr in range(ROW_UNROLL):
            r = i * ROW_UNROLL + rr
            for j in range(D_MODEL // 16):
                dst[r, pl.ds(j * 16, 16)] = src[r, pl.ds(j * 16, 16)] * SCALE
        return carry

    lax.fori_loop(0, C // ROW_UNROLL, body, 0, unroll=False)


def _make_gather(n_total):
    n_w = n_total // NW
    steps = n_w // C
    groups = steps // NBUF
    mesh = plsc.VectorSubcoreMesh(core_axis_name="c", subcore_axis_name="s")

    def body(table, idx_hbm, out_hbm, idx_v, in_bufs, out_bufs, gsems, osems):
        wid = lax.axis_index("s") * NC + lax.axis_index("c")
        base = wid * n_w
        pltpu.sync_copy(idx_hbm.at[pl.ds(base, n_w)], idx_v)

        def fire_gather(s, b):
            pltpu.async_copy(
                table.at[idx_v.at[pl.ds(s * C, C)]], in_bufs.at[b], gsems.at[b]
            )

        def wait_gather(b):
            pltpu.make_async_copy(
                table.at[pl.ds(0, C)], in_bufs.at[b], gsems.at[b]
            ).wait()

        def fire_scatter(s, b):
            pltpu.async_copy(
                out_bufs.at[b], out_hbm.at[pl.ds(base + s * C, C)], osems.at[b]
            )

        def wait_scatter(b):
            pltpu.make_async_copy(
                out_bufs.at[b], out_hbm.at[pl.ds(0, C)], osems.at[b]
            ).wait()

        for b in range(NBUF):
            fire_gather(b, b)

        for b in range(NBUF):
            wait_gather(b)
            _scale_chunk(in_bufs.at[b], out_bufs.at[b])
            fire_gather(b + NBUF, b)
            fire_scatter(b, b)

        def group(g, carry):
            for b in range(NBUF):
                s = g * NBUF + b
                wait_gather(b)
                wait_scatter(b)
                _scale_chunk(in_bufs.at[b], out_bufs.at[b])
                fire_gather(s + NBUF, b)
                fire_scatter(s, b)
            return carry

        lax.fori_loop(1, groups - 1, group, 0, unroll=False)

        for b in range(NBUF):
            s = (groups - 1) * NBUF + b
            wait_gather(b)
            wait_scatter(b)
            _scale_chunk(in_bufs.at[b], out_bufs.at[b])
            fire_scatter(s, b)

        for b in range(NBUF):
            wait_scatter(b)

    grid_kernel = pl.kernel(
        body,
        out_type=jax.ShapeDtypeStruct((n_total, D_MODEL), jnp.float32),
        mesh=mesh,
        compiler_params=pltpu.CompilerParams(use_tc_tiling_on_sc=False),
        scratch_types=[
            pltpu.VMEM((n_w,), jnp.int32),
            pltpu.VMEM((NBUF, C, D_MODEL), jnp.float32),
            pltpu.VMEM((NBUF, C, D_MODEL), jnp.float32),
            pltpu.SemaphoreType.DMA((NBUF,)),
            pltpu.SemaphoreType.DMA((NBUF,)),
        ],
    )
    return grid_kernel


def kernel(x, emb_weight):
    b, s = x.shape
    n_total = b * s
    flat_idx = x.reshape(n_total).astype(jnp.int32)
    out = _make_gather(n_total)(emb_weight, flat_idx)
    return out.reshape(b, s, D_MODEL)
